# baseline (speedup 1.0000x reference)
import functools

import jax
import jax.numpy as jnp
from jax import lax
from jax.experimental import pallas as pl
from jax.experimental.pallas import tpu as pltpu
from jax.experimental.pallas import tpu_sc as plsc

BATCH = 16384
FIELDS = 26
D = 64
B = BATCH * FIELDS
NW = 32
CHUNK = 128
ROWS_PER_W = B // NW
CH_PER_W = ROWS_PER_W // CHUNK
NBUF = 2


def _swish_chunk(src, dst):
    def row_body(j, c):
        for t in range(D // 16):
            v = src[j, pl.ds(t * 16, 16)]
            dst[j, pl.ds(t * 16, 16)] = v / (1.0 + jnp.exp(-v))
        return c

    lax.fori_loop(0, CHUNK, row_body, 0, unroll=4)


@functools.partial(
    pl.kernel,
    out_type=jax.ShapeDtypeStruct((B, D), jnp.float32),
    mesh=plsc.VectorSubcoreMesh(core_axis_name="c", subcore_axis_name="s"),
    scratch_types=[
        pltpu.VMEM((CH_PER_W, CHUNK), jnp.int32),
        [pltpu.VMEM((CHUNK, D), jnp.float32) for _ in range(NBUF)],
        [pltpu.VMEM((CHUNK, D), jnp.float32) for _ in range(NBUF)],
        [pltpu.SemaphoreType.DMA for _ in range(NBUF)],
        [pltpu.SemaphoreType.DMA for _ in range(NBUF)],
    ],
    compiler_params=pltpu.CompilerParams(use_tc_tiling_on_sc=False),
)
def _emb_swish(idx_hbm, table_hbm, out_hbm, idx_v, gbuf, sbuf, gsem, ssem):
    wid = lax.axis_index("s") * 2 + lax.axis_index("c")
    base_chunk = wid * CH_PER_W
    pltpu.sync_copy(idx_hbm.at[pl.ds(base_chunk, CH_PER_W)], idx_v)

    def gather(g, b):
        return pltpu.make_async_copy(table_hbm.at[idx_v.at[g]], gbuf[b], gsem[b])

    def store(g, b):
        return pltpu.make_async_copy(
            sbuf[b], out_hbm.at[pl.ds((base_chunk + g) * CHUNK, CHUNK)], ssem[b]
        )

    for b in range(NBUF):
        gather(b, b).start()

    def outer(i, carry):
        for b in range(NBUF):
            g = i * NBUF + b
            gather(g, b).wait()

            @pl.when(g >= NBUF)
            def _():
                store(g, b).wait()

            _swish_chunk(gbuf[b], sbuf[b])

            @pl.when(g + NBUF < CH_PER_W)
            def _():
                gather(g + NBUF, b).start()

            store(g, b).start()
        return carry

    lax.fori_loop(0, CH_PER_W // NBUF, outer, 0)
    for b in range(NBUF):
        store(0, b).wait()


def kernel(x, emb_weight):
    idx = x.astype(jnp.int32).reshape(CH_PER_W * NW, CHUNK)
    out = _emb_swish(idx, emb_weight)
    return out.reshape(BATCH, FIELDS, D)


# speedup vs baseline: 1.9243x; 1.9243x over previous
import functools

import jax
import jax.numpy as jnp
from jax import lax
from jax.experimental import pallas as pl
from jax.experimental.pallas import tpu as pltpu
from jax.experimental.pallas import tpu_sc as plsc

BATCH = 16384
FIELDS = 26
D = 64
B = BATCH * FIELDS
NW = 32
CHUNK = 128
ROWS_PER_W = B // NW
CH_PER_W = ROWS_PER_W // CHUNK
NBUF = 2


def _swish_chunk(src, dst):
    def row_body(j, c):
        for t in range(D // 16):
            v = src[j, pl.ds(t * 16, 16)]
            dst[j, pl.ds(t * 16, 16)] = v / (1.0 + jnp.exp(-v))
        return c

    lax.fori_loop(0, CHUNK, row_body, 0)


@functools.partial(
    pl.kernel,
    out_type=jax.ShapeDtypeStruct((B, D), jnp.float32),
    mesh=plsc.VectorSubcoreMesh(core_axis_name="c", subcore_axis_name="s"),
    scratch_types=[
        pltpu.VMEM((CH_PER_W, CHUNK), jnp.int32),
        [pltpu.VMEM((CHUNK, D), jnp.float32) for _ in range(NBUF)],
        [pltpu.VMEM((CHUNK, D), jnp.float32) for _ in range(NBUF)],
        [pltpu.SemaphoreType.DMA for _ in range(NBUF)],
        [pltpu.SemaphoreType.DMA for _ in range(NBUF)],
    ],
    compiler_params=pltpu.CompilerParams(use_tc_tiling_on_sc=False),
)
def _emb_swish(idx_hbm, table_hbm, out_hbm, idx_v, gbuf, sbuf, gsem, ssem):
    wid = lax.axis_index("s") * 2 + lax.axis_index("c")
    base_chunk = wid * CH_PER_W
    pltpu.sync_copy(idx_hbm.at[pl.ds(base_chunk, CH_PER_W)], idx_v)

    def gather(g, b):
        return pltpu.make_async_copy(table_hbm.at[idx_v.at[g]], gbuf[b], gsem[b])

    def store(g, b):
        return pltpu.make_async_copy(
            sbuf[b], out_hbm.at[pl.ds((base_chunk + g) * CHUNK, CHUNK)], ssem[b]
        )

    for b in range(NBUF):
        gather(b, b).start()

    def outer(i, carry):
        for b in range(NBUF):
            g = i * NBUF + b
            gather(g, b).wait()

            @pl.when(g >= NBUF)
            def _():
                store(g, b).wait()

            _swish_chunk(gbuf[b], sbuf[b])

            @pl.when(g + NBUF < CH_PER_W)
            def _():
                gather(g + NBUF, b).start()

            store(g, b).start()
        return carry

    lax.fori_loop(0, CH_PER_W // NBUF, outer, 0)
    for b in range(NBUF):
        store(0, b).wait()


def kernel(x, emb_weight):
    idx = x.astype(jnp.int32).reshape(CH_PER_W * NW, CHUNK)
    out = _emb_swish(idx, emb_weight)
    return out.reshape(BATCH, FIELDS, D)
